# baseline (speedup 1.0000x reference)
import jax
import jax.numpy as jnp
from jax.experimental import pallas as pl
from jax.experimental.pallas import tpu as pltpu


def _round_up(x, m):
    return ((x + m - 1) // m) * m


def _make_icm_kernel(S, A, CB, n_chunks, nbuf):
    def _icm_kernel(
        state_hbm, next_state_hbm, action_hbm,
        w1p_ref, b1p_ref, w2p_ref, b2p_ref, w3p_ref, b3p_ref,
        w1i_ref, b1i_ref, w2i_ref, b2i_ref, w3i_ref, b3i_ref,
        ns_out_hbm, ap_out_hbm,
        s_buf, n_buf, a_buf, y1_buf, y2_buf, sem_in, sem_out,
    ):
        bf16 = jnp.bfloat16

        def start_in(step):
            slot = step % nbuf
            sl = pl.ds(step * CB, CB)
            pltpu.make_async_copy(
                state_hbm.at[sl], s_buf.at[slot], sem_in.at[slot, 0]).start()
            pltpu.make_async_copy(
                next_state_hbm.at[sl], n_buf.at[slot], sem_in.at[slot, 1]).start()
            pltpu.make_async_copy(
                action_hbm.at[sl], a_buf.at[slot], sem_in.at[slot, 2]).start()

        def wait_in(step):
            slot = step % nbuf
            pltpu.make_async_copy(
                s_buf.at[slot], s_buf.at[slot], sem_in.at[slot, 0]).wait()
            pltpu.make_async_copy(
                n_buf.at[slot], n_buf.at[slot], sem_in.at[slot, 1]).wait()
            pltpu.make_async_copy(
                a_buf.at[slot], a_buf.at[slot], sem_in.at[slot, 2]).wait()

        def start_out(step):
            slot = step % nbuf
            sl = pl.ds(step * CB, CB)
            pltpu.make_async_copy(
                y1_buf.at[slot], ns_out_hbm.at[sl], sem_out.at[slot, 0]).start()
            pltpu.make_async_copy(
                y2_buf.at[slot], ap_out_hbm.at[sl], sem_out.at[slot, 1]).start()

        def wait_out(step):
            slot = step % nbuf
            pltpu.make_async_copy(
                y1_buf.at[slot], y1_buf.at[slot], sem_out.at[slot, 0]).wait()
            pltpu.make_async_copy(
                y2_buf.at[slot], y2_buf.at[slot], sem_out.at[slot, 1]).wait()

        def relu_bf16(acc32, b_ref):
            z = acc32.astype(bf16) + b_ref[...].astype(bf16)
            return jnp.maximum(z, jnp.zeros_like(z))

        def compute(slot):
            s = s_buf[slot].astype(bf16)
            ns = n_buf[slot].astype(bf16)
            a = a_buf[slot].astype(bf16)

            h = (jnp.dot(s, w1p_ref[:S].astype(bf16),
                         preferred_element_type=jnp.float32)
                 + jnp.dot(a, w1p_ref[S:].astype(bf16),
                           preferred_element_type=jnp.float32))
            g = (jnp.dot(s, w1i_ref[:S].astype(bf16),
                         preferred_element_type=jnp.float32)
                 + jnp.dot(ns, w1i_ref[S:].astype(bf16),
                           preferred_element_type=jnp.float32))
            h = relu_bf16(h, b1p_ref)
            g = relu_bf16(g, b1i_ref)
            h = jnp.dot(h, w2p_ref[...].astype(bf16),
                        preferred_element_type=jnp.float32)
            g = jnp.dot(g, w2i_ref[...].astype(bf16),
                        preferred_element_type=jnp.float32)
            h = relu_bf16(h, b2p_ref)
            g = relu_bf16(g, b2i_ref)
            y1_buf[slot] = (
                jnp.dot(h, w3p_ref[...].astype(bf16),
                        preferred_element_type=jnp.float32)
                + b3p_ref[...])
            y2_buf[slot] = (
                jnp.dot(g, w3i_ref[...].astype(bf16),
                        preferred_element_type=jnp.float32)
                + b3i_ref[...])

        for step in range(min(nbuf - 1, n_chunks)):
            start_in(step)
        for step in range(n_chunks):
            if step + nbuf - 1 < n_chunks:
                start_in(step + nbuf - 1)
            wait_in(step)
            if step >= nbuf:
                wait_out(step - nbuf)
            compute(step % nbuf)
            start_out(step)
        for step in range(max(n_chunks - nbuf, 0), n_chunks):
            wait_out(step)

    return _icm_kernel


def kernel(state, next_state, action,
           w1p, b1p, w2p, b2p, w3p, b3p,
           w1i, b1i, w2i, b2i, w3i, b3i,
           *, chunk_b=2048, nbuf=2):
    B, S = state.shape
    A = action.shape[1]

    CB = min(chunk_b, _round_up(B, 8))
    b_pad = _round_up(B, CB)
    if b_pad != B:
        pad = ((0, b_pad - B), (0, 0))
        state = jnp.pad(state, pad)
        next_state = jnp.pad(next_state, pad)
        action = jnp.pad(action, pad)
    n_chunks = b_pad // CB

    param_arrays = [w1p, b1p, w2p, b2p, w3p, b3p,
                    w1i, b1i, w2i, b2i, w3i, b3i]

    H1 = w1p.shape[1]
    H2 = w2p.shape[1]
    flops = 2 * b_pad * ((S + A) * H1 + 2 * S * H1 + 2 * H1 * H2
                         + H2 * S + H2 * A)
    bytes_accessed = (4 * b_pad * (2 * S + A + S + A)
                      + 4 * sum(int(p.size) for p in param_arrays))

    f32 = jnp.float32
    ns_pred, a_pred = pl.pallas_call(
        _make_icm_kernel(S, A, CB, n_chunks, nbuf),
        out_shape=(jax.ShapeDtypeStruct((b_pad, S), f32),
                   jax.ShapeDtypeStruct((b_pad, A), f32)),
        in_specs=([pl.BlockSpec(memory_space=pl.ANY)] * 3
                  + [pl.BlockSpec(memory_space=pltpu.VMEM)] * 12),
        out_specs=(pl.BlockSpec(memory_space=pl.ANY),
                   pl.BlockSpec(memory_space=pl.ANY)),
        scratch_shapes=[
            pltpu.VMEM((nbuf, CB, S), f32),
            pltpu.VMEM((nbuf, CB, S), f32),
            pltpu.VMEM((nbuf, CB, A), f32),
            pltpu.VMEM((nbuf, CB, S), f32),
            pltpu.VMEM((nbuf, CB, A), f32),
            pltpu.SemaphoreType.DMA((nbuf, 3)),
            pltpu.SemaphoreType.DMA((nbuf, 2)),
        ],
        cost_estimate=pl.CostEstimate(
            flops=flops, transcendentals=0, bytes_accessed=bytes_accessed),
    )(state, next_state, action, *param_arrays)

    if b_pad != B:
        ns_pred, a_pred = ns_pred[:B], a_pred[:B]
    return ns_pred, a_pred


# speedup vs baseline: 1.0440x; 1.0440x over previous
import jax
import jax.numpy as jnp
from jax.experimental import pallas as pl
from jax.experimental.pallas import tpu as pltpu


def _round_up(x, m):
    return ((x + m - 1) // m) * m


def _make_icm_kernel(S):
    def _icm_kernel(
        state_ref, next_state_ref, action_ref,
        w1p_ref, b1p_ref, w2p_ref, b2p_ref, w3p_ref, b3p_ref,
        w1i_ref, b1i_ref, w2i_ref, b2i_ref, w3i_ref, b3i_ref,
        ns_out_ref, ap_out_ref,
    ):
        bf16 = jnp.bfloat16
        s = state_ref[...].astype(bf16)
        ns = next_state_ref[...].astype(bf16)
        a = action_ref[...].astype(bf16)

        def relu_bf16(acc32, b_ref):
            z = acc32.astype(bf16) + b_ref[...].astype(bf16)
            return jnp.maximum(z, jnp.zeros_like(z))

        h = (jnp.dot(s, w1p_ref[:S].astype(bf16),
                     preferred_element_type=jnp.float32)
             + jnp.dot(a, w1p_ref[S:].astype(bf16),
                       preferred_element_type=jnp.float32))
        g = (jnp.dot(s, w1i_ref[:S].astype(bf16),
                     preferred_element_type=jnp.float32)
             + jnp.dot(ns, w1i_ref[S:].astype(bf16),
                       preferred_element_type=jnp.float32))
        h = relu_bf16(h, b1p_ref)
        g = relu_bf16(g, b1i_ref)
        h = jnp.dot(h, w2p_ref[...].astype(bf16),
                    preferred_element_type=jnp.float32)
        g = jnp.dot(g, w2i_ref[...].astype(bf16),
                    preferred_element_type=jnp.float32)
        h = relu_bf16(h, b2p_ref)
        g = relu_bf16(g, b2i_ref)
        ns_out_ref[...] = (
            jnp.dot(h, w3p_ref[...].astype(bf16),
                    preferred_element_type=jnp.float32)
            + b3p_ref[...])
        ap_out_ref[...] = (
            jnp.dot(g, w3i_ref[...].astype(bf16),
                    preferred_element_type=jnp.float32)
            + b3i_ref[...])

    return _icm_kernel


def kernel(state, next_state, action,
           w1p, b1p, w2p, b2p, w3p, b3p,
           w1i, b1i, w2i, b2i, w3i, b3i,
           *, tile_b=1024):
    B, S = state.shape
    A = action.shape[1]

    tile_b = min(tile_b, _round_up(B, 8))
    b_pad = _round_up(B, tile_b)
    if b_pad != B:
        pad = ((0, b_pad - B), (0, 0))
        state = jnp.pad(state, pad)
        next_state = jnp.pad(next_state, pad)
        action = jnp.pad(action, pad)
    grid = (b_pad // tile_b,)

    param_arrays = [w1p, b1p, w2p, b2p, w3p, b3p,
                    w1i, b1i, w2i, b2i, w3i, b3i]

    def batch_spec(n):
        return pl.BlockSpec((tile_b, n), lambda i: (i, 0))

    def param_spec(shape):
        return pl.BlockSpec(shape, lambda i: (0, 0))

    in_specs = ([batch_spec(S), batch_spec(S), batch_spec(A)]
                + [param_spec(tuple(p.shape)) for p in param_arrays])

    H1 = w1p.shape[1]
    H2 = w2p.shape[1]
    flops = 2 * b_pad * ((S + A) * H1 + 2 * S * H1 + 2 * H1 * H2
                         + H2 * S + H2 * A)
    bytes_accessed = (4 * b_pad * (2 * S + A + S + A)
                      + 4 * sum(int(p.size) for p in param_arrays))

    ns_pred, a_pred = pl.pallas_call(
        _make_icm_kernel(S),
        out_shape=(jax.ShapeDtypeStruct((b_pad, S), jnp.float32),
                   jax.ShapeDtypeStruct((b_pad, A), jnp.float32)),
        grid=grid,
        in_specs=in_specs,
        out_specs=(pl.BlockSpec((tile_b, S), lambda i: (i, 0)),
                   pl.BlockSpec((tile_b, A), lambda i: (i, 0))),
        compiler_params=pltpu.CompilerParams(
            dimension_semantics=("parallel",)),
        cost_estimate=pl.CostEstimate(
            flops=flops, transcendentals=0, bytes_accessed=bytes_accessed),
    )(state, next_state, action, *param_arrays)

    if b_pad != B:
        ns_pred, a_pred = ns_pred[:B], a_pred[:B]
    return ns_pred, a_pred
